# baseline (speedup 1.0000x reference)
import jax
import jax.numpy as jnp
from jax.experimental import pallas as pl


def _body(x_ref, ww_ref, wc_ref, o_ref):
    pos = ww_ref[...][None, :, :] + wc_ref[...][:, None, :]
    o_ref[...] = x_ref[...] + pos[:, :, None, :]


def kernel(input_embeddings, W_word, W_char):
    C, W, B, D = input_embeddings.shape
    BC = 8
    BW = 256
    return pl.pallas_call(
        _body,
        grid=(C // BC, W // BW),
        in_specs=[
            pl.BlockSpec((BC, BW, B, D), lambda i, j: (i, j, 0, 0)),
            pl.BlockSpec((BW, D), lambda i, j: (j, 0)),
            pl.BlockSpec((BC, D), lambda i, j: (i, 0)),
        ],
        out_specs=pl.BlockSpec((BC, BW, B, D), lambda i, j: (i, j, 0, 0)),
        out_shape=jax.ShapeDtypeStruct((C, W, B, D), jnp.float32),
    )(input_embeddings, W_word, W_char)


# speedup vs baseline: 6.8256x; 6.8256x over previous
import jax
import jax.numpy as jnp
from jax.experimental import pallas as pl


def _body(x_ref, ww_ref, wc_ref, o_ref):
    o_ref[...] = (
        x_ref[...]
        + ww_ref[...][None, None, :, :]
        + wc_ref[...][:, None, :, None]
    )


def kernel(input_embeddings, W_word, W_char):
    C, W, B, D = input_embeddings.shape
    x_t = jnp.transpose(input_embeddings, (0, 2, 3, 1))
    ww_t = W_word.T

    BC, BW = 8, 256
    out_t = pl.pallas_call(
        _body,
        grid=(C // BC, W // BW),
        in_specs=[
            pl.BlockSpec((BC, B, D, BW), lambda i, j: (i, 0, 0, j)),
            pl.BlockSpec((D, BW), lambda i, j: (0, j)),
            pl.BlockSpec((BC, D), lambda i, j: (i, 0)),
        ],
        out_specs=pl.BlockSpec((BC, B, D, BW), lambda i, j: (i, 0, 0, j)),
        out_shape=jax.ShapeDtypeStruct((C, B, D, W), jnp.float32),
    )(x_t, ww_t, W_char)
    return jnp.transpose(out_t, (0, 3, 1, 2))


# speedup vs baseline: 7.6497x; 1.1207x over previous
import jax
import jax.numpy as jnp
from jax.experimental import pallas as pl


def _body(x_ref, ww_ref, wc_ref, o_ref):
    o_ref[...] = (
        x_ref[...]
        + ww_ref[...][None, None, :, :]
        + wc_ref[...][:, None, :, None]
    )


def kernel(input_embeddings, W_word, W_char):
    C, W, B, D = input_embeddings.shape
    x_t = jnp.transpose(input_embeddings, (0, 2, 3, 1))
    ww_t = W_word.T

    BC = 8
    out_t = pl.pallas_call(
        _body,
        grid=(C // BC,),
        in_specs=[
            pl.BlockSpec((BC, B, D, W), lambda i: (i, 0, 0, 0)),
            pl.BlockSpec((D, W), lambda i: (0, 0)),
            pl.BlockSpec((BC, D), lambda i: (i, 0)),
        ],
        out_specs=pl.BlockSpec((BC, B, D, W), lambda i: (i, 0, 0, 0)),
        out_shape=jax.ShapeDtypeStruct((C, B, D, W), jnp.float32),
    )(x_t, ww_t, W_char)
    return jnp.transpose(out_t, (0, 3, 1, 2))
